# baseline (speedup 1.0000x reference)
import functools

import jax
import jax.numpy as jnp
from jax import lax
from jax.experimental import pallas as pl
from jax.experimental.pallas import tpu as pltpu
from jax.experimental.pallas import tpu_sc as plsc

NUM_CORES = 2
NUM_SUBCORES = 16
NW = NUM_CORES * NUM_SUBCORES

B = 16384
J = 26
DIM = 64
N_TOTAL = B * J
IDX_W = 128
CHUNKS = N_TOTAL // (NW * IDX_W)
BLK_PER_ROW = B // IDX_W


def _gather_body(table_hbm, idx_hbm, out_hbm, idx_v, ga, gb, ta, tb,
                 gsa, gsb, wsa, wsb):
    wid = lax.axis_index("s") * NUM_CORES + lax.axis_index("c")
    pltpu.sync_copy(idx_hbm.at[pl.ds(wid * CHUNKS, CHUNKS)], idx_v)
    base = wid * CHUNKS

    rows = [lax.iota(jnp.int32, 16) + 16 * k for k in range(8)]

    def fire_g(buf, sem, t):
        pltpu.async_copy(table_hbm.at[idx_v.at[t]], buf, sem)

    def wait_g(buf, sem, t):
        pltpu.make_async_copy(table_hbm.at[idx_v.at[t]], buf, sem).wait()

    def out_slice(t):
        flat = base + t
        j = flat // BLK_PER_ROW
        b0 = (flat % BLK_PER_ROW) * IDX_W
        return out_hbm.at[j, :, pl.ds(b0, IDX_W)]

    def fire_w(buf, sem, t):
        pltpu.async_copy(buf, out_slice(t), sem)

    def wait_w(buf, sem, t):
        pltpu.make_async_copy(buf, out_slice(t), sem).wait()

    def transpose(gbuf, tbuf):
        def body(d, carry):
            cols = jnp.full((16,), d, jnp.int32)
            for k in range(8):
                v = plsc.load_gather(gbuf, [rows[k], cols])
                tbuf[d, pl.ds(16 * k, 16)] = v
            return carry
        lax.fori_loop(0, DIM, body, 0, unroll=4)

    fire_g(ga, gsa, 0)

    def step(T, carry):
        t = 2 * T
        fire_g(gb, gsb, t + 1)
        wait_g(ga, gsa, t)
        transpose(ga, ta)

        @pl.when(T > 0)
        def _():
            wait_w(ta, wsa, t - 2)
        fire_w(ta, wsa, t)

        @pl.when(T < CHUNKS // 2 - 1)
        def _():
            fire_g(ga, gsa, t + 2)
        wait_g(gb, gsb, t + 1)
        transpose(gb, tb)

        @pl.when(T > 0)
        def _():
            wait_w(tb, wsb, t - 1)
        fire_w(tb, wsb, t + 1)
        return carry

    lax.fori_loop(0, CHUNKS // 2, step, 0)
    wait_w(ta, wsa, CHUNKS - 2)
    wait_w(tb, wsb, CHUNKS - 1)


@jax.jit
def _gather(x2d, table):
    mesh = plsc.VectorSubcoreMesh(core_axis_name="c", subcore_axis_name="s")
    k = pl.kernel(
        _gather_body,
        mesh=mesh,
        out_type=jax.ShapeDtypeStruct((J, DIM, B), jnp.float32),
        scratch_types=[
            pltpu.VMEM((CHUNKS, IDX_W), jnp.int32),
            pltpu.VMEM((IDX_W, DIM), jnp.float32),
            pltpu.VMEM((IDX_W, DIM), jnp.float32),
            pltpu.VMEM((DIM, IDX_W), jnp.float32),
            pltpu.VMEM((DIM, IDX_W), jnp.float32),
            pltpu.SemaphoreType.DMA,
            pltpu.SemaphoreType.DMA,
            pltpu.SemaphoreType.DMA,
            pltpu.SemaphoreType.DMA,
        ],
        compiler_params=pltpu.CompilerParams(
            use_tc_tiling_on_sc=False, needs_layout_passes=False),
    )
    return k(table, x2d)


def kernel(x, table):
    x2d = jnp.swapaxes(x, 0, 1).reshape(NW * CHUNKS, IDX_W)
    out = _gather(x2d, table)
    return jnp.transpose(out, (2, 0, 1))


# speedup vs baseline: 1.4128x; 1.4128x over previous
import jax
import jax.numpy as jnp
from jax import lax
from jax.experimental import pallas as pl
from jax.experimental.pallas import tpu as pltpu
from jax.experimental.pallas import tpu_sc as plsc

NUM_CORES = 2
NUM_SUBCORES = 16
NW = NUM_CORES * NUM_SUBCORES

B = 16384
J = 26
DIM = 64
N_TOTAL = B * J
IDX_W = 128
CHUNKS = N_TOTAL // (NW * IDX_W)
NBUF = 4


def _gather_body(table_hbm, idx_hbm, out_hbm, idx_v, g0, g1, g2, g3,
                 gs0, gs1, gs2, gs3, ws0, ws1, ws2, ws3):
    bufs = [g0, g1, g2, g3]
    gsems = [gs0, gs1, gs2, gs3]
    wsems = [ws0, ws1, ws2, ws3]

    wid = lax.axis_index("s") * NUM_CORES + lax.axis_index("c")
    pltpu.sync_copy(idx_hbm.at[pl.ds(wid * CHUNKS, CHUNKS)], idx_v)
    base = wid * CHUNKS

    def fire_g(b, t):
        pltpu.async_copy(table_hbm.at[idx_v.at[t]], bufs[b], gsems[b])

    def wait_g(b, t):
        pltpu.make_async_copy(
            table_hbm.at[idx_v.at[t]], bufs[b], gsems[b]).wait()

    def out_slice(t):
        return out_hbm.at[pl.ds((base + t) * IDX_W, IDX_W)]

    def fire_w(b, t):
        pltpu.async_copy(bufs[b], out_slice(t), wsems[b])

    def wait_w(b, t):
        pltpu.make_async_copy(bufs[b], out_slice(t), wsems[b]).wait()

    for t in range(NBUF - 1):
        fire_g(t, t)

    def group(G, carry):
        for b in range(NBUF):
            t = NBUF * G + b
            wait_g(b, t)
            fire_w(b, t)
            bn = (b + NBUF - 1) % NBUF

            @pl.when(t >= 1)
            def _():
                wait_w(bn, t - 1)

            @pl.when(t + NBUF - 1 < CHUNKS)
            def _():
                fire_g(bn, t + NBUF - 1)
        return carry

    lax.fori_loop(0, CHUNKS // NBUF, group, 0)
    wait_w((CHUNKS - 1) % NBUF, CHUNKS - 1)


@jax.jit
def _gather(x2d, table):
    mesh = plsc.VectorSubcoreMesh(core_axis_name="c", subcore_axis_name="s")
    k = pl.kernel(
        _gather_body,
        mesh=mesh,
        out_type=jax.ShapeDtypeStruct((N_TOTAL, DIM), jnp.float32),
        scratch_types=[
            pltpu.VMEM((CHUNKS, IDX_W), jnp.int32),
            pltpu.VMEM((IDX_W, DIM), jnp.float32),
            pltpu.VMEM((IDX_W, DIM), jnp.float32),
            pltpu.VMEM((IDX_W, DIM), jnp.float32),
            pltpu.VMEM((IDX_W, DIM), jnp.float32),
            pltpu.SemaphoreType.DMA,
            pltpu.SemaphoreType.DMA,
            pltpu.SemaphoreType.DMA,
            pltpu.SemaphoreType.DMA,
            pltpu.SemaphoreType.DMA,
            pltpu.SemaphoreType.DMA,
            pltpu.SemaphoreType.DMA,
            pltpu.SemaphoreType.DMA,
        ],
        compiler_params=pltpu.CompilerParams(
            use_tc_tiling_on_sc=False, needs_layout_passes=False),
    )
    return k(table, x2d)


def kernel(x, table):
    x2d = x.reshape(NW * CHUNKS, IDX_W)
    out = _gather(x2d, table)
    return out.reshape(B, J, DIM)


# speedup vs baseline: 1.4130x; 1.0002x over previous
import jax
import jax.numpy as jnp
from jax import lax
from jax.experimental import pallas as pl
from jax.experimental.pallas import tpu as pltpu
from jax.experimental.pallas import tpu_sc as plsc

NUM_CORES = 2
NUM_SUBCORES = 16
NW = NUM_CORES * NUM_SUBCORES

B = 16384
J = 26
DIM = 64
N_TOTAL = B * J
IDX_W = 256
CHUNKS = N_TOTAL // (NW * IDX_W)
NBUF = 4


def _gather_body(table_hbm, idx_hbm, out_hbm, idx_v, g0, g1, g2, g3,
                 gs0, gs1, gs2, gs3, ws0, ws1, ws2, ws3):
    bufs = [g0, g1, g2, g3]
    gsems = [gs0, gs1, gs2, gs3]
    wsems = [ws0, ws1, ws2, ws3]

    wid = lax.axis_index("s") * NUM_CORES + lax.axis_index("c")
    pltpu.sync_copy(idx_hbm.at[pl.ds(wid * CHUNKS, CHUNKS)], idx_v)
    base = wid * CHUNKS

    def fire_g(b, t):
        pltpu.async_copy(table_hbm.at[idx_v.at[t]], bufs[b], gsems[b])

    def wait_g(b, t):
        pltpu.make_async_copy(
            table_hbm.at[idx_v.at[t]], bufs[b], gsems[b]).wait()

    def out_slice(t):
        return out_hbm.at[pl.ds((base + t) * IDX_W, IDX_W)]

    def fire_w(b, t):
        pltpu.async_copy(bufs[b], out_slice(t), wsems[b])

    def wait_w(b, t):
        pltpu.make_async_copy(bufs[b], out_slice(t), wsems[b]).wait()

    for t in range(NBUF - 1):
        fire_g(t, t)

    def group(G, carry):
        for b in range(NBUF):
            t = NBUF * G + b
            wait_g(b, t)
            fire_w(b, t)
            bn = (b + NBUF - 1) % NBUF

            @pl.when(t >= 1)
            def _():
                wait_w(bn, t - 1)

            @pl.when(t + NBUF - 1 < CHUNKS)
            def _():
                fire_g(bn, t + NBUF - 1)
        return carry

    lax.fori_loop(0, CHUNKS // NBUF, group, 0)
    wait_w((CHUNKS - 1) % NBUF, CHUNKS - 1)


@jax.jit
def _gather(x2d, table):
    mesh = plsc.VectorSubcoreMesh(core_axis_name="c", subcore_axis_name="s")
    k = pl.kernel(
        _gather_body,
        mesh=mesh,
        out_type=jax.ShapeDtypeStruct((N_TOTAL, DIM), jnp.float32),
        scratch_types=[
            pltpu.VMEM((CHUNKS, IDX_W), jnp.int32),
            pltpu.VMEM((IDX_W, DIM), jnp.float32),
            pltpu.VMEM((IDX_W, DIM), jnp.float32),
            pltpu.VMEM((IDX_W, DIM), jnp.float32),
            pltpu.VMEM((IDX_W, DIM), jnp.float32),
            pltpu.SemaphoreType.DMA,
            pltpu.SemaphoreType.DMA,
            pltpu.SemaphoreType.DMA,
            pltpu.SemaphoreType.DMA,
            pltpu.SemaphoreType.DMA,
            pltpu.SemaphoreType.DMA,
            pltpu.SemaphoreType.DMA,
            pltpu.SemaphoreType.DMA,
        ],
        compiler_params=pltpu.CompilerParams(
            use_tc_tiling_on_sc=False, needs_layout_passes=False),
    )
    return k(table, x2d)


def kernel(x, table):
    x2d = x.reshape(NW * CHUNKS, IDX_W)
    out = _gather(x2d, table)
    return out.reshape(B, J, DIM)
